# baseline (speedup 1.0000x reference)
import functools

import jax
import jax.numpy as jnp
from jax import lax
from jax.experimental import pallas as pl
from jax.experimental.pallas import tpu as pltpu
from jax.experimental.pallas import tpu_sc as plsc

_NC = 2
_NS = 16
_NW = _NC * _NS
_L = 16
_CHUNK = 128
_CPB = 8


def _fill_vmem(ref, rows, cols, value):
  vec = jnp.full((_L,), value, jnp.float32)

  def body(i, c):
    for j in range(cols // _L):
      ref[i, pl.ds(j * _L, _L)] = vec
    return c

  lax.fori_loop(0, rows, body, 0)


def _zero_shared_rows(src_v, acc_sh, rbase, rows):
  nfull, rem = divmod(rows, _CHUNK)
  for k in range(nfull):
    pltpu.sync_copy(src_v, acc_sh.at[pl.ds(rbase + k * _CHUNK, _CHUNK)])
  if rem:
    pltpu.sync_copy(
        src_v.at[pl.ds(0, rem)],
        acc_sh.at[pl.ds(rbase + nfull * _CHUNK, rem)],
    )


def _make_deg_kernel(n_pad, blocks_per_tile):
  rows = n_pad // _NS

  @functools.partial(
      pl.kernel,
      out_type=jax.ShapeDtypeStruct((_NC, n_pad, _L), jnp.float32),
      mesh=plsc.VectorSubcoreMesh(core_axis_name="c", subcore_axis_name="s"),
      compiler_params=pltpu.CompilerParams(use_tc_tiling_on_sc=False),
      scratch_types=[
          pltpu.VMEM((_CPB, _CHUNK), jnp.int32),
          pltpu.VMEM((_CHUNK, _L), jnp.float32),
          pltpu.VMEM_SHARED((n_pad, _L), jnp.float32),
      ],
  )
  def deg_kernel(dst_hbm, out_hbm, dst_i, val_v, acc_sh):
    cid = lax.axis_index("c")
    sid = lax.axis_index("s")
    wid = sid * _NC + cid
    rbase = sid * rows

    _fill_vmem(val_v, _CHUNK, _L, 0.0)
    _zero_shared_rows(val_v, acc_sh, rbase, rows)
    plsc.subcore_barrier()

    _fill_vmem(val_v, _CHUNK, _L, 1.0)
    bbase = wid * blocks_per_tile

    def outer(t, c):
      crow = (bbase + t) * _CPB
      pltpu.sync_copy(dst_hbm.at[pl.ds(crow, _CPB)], dst_i)
      for j in range(_CPB):
        pltpu.sync_copy(val_v, acc_sh.at[dst_i.at[j]], add=True)
      return c

    lax.fori_loop(0, blocks_per_tile, outer, 0)
    plsc.subcore_barrier()
    pltpu.sync_copy(
        acc_sh.at[pl.ds(rbase, rows)], out_hbm.at[cid, pl.ds(rbase, rows)]
    )

  return deg_kernel


def _make_agg_kernel(n_pad, d, blocks_per_tile):
  rows = n_pad // _NS

  @functools.partial(
      pl.kernel,
      out_type=jax.ShapeDtypeStruct((_NC, n_pad, d), jnp.float32),
      mesh=plsc.VectorSubcoreMesh(core_axis_name="c", subcore_axis_name="s"),
      compiler_params=pltpu.CompilerParams(use_tc_tiling_on_sc=False),
      scratch_types=[
          pltpu.VMEM((_CPB, _CHUNK), jnp.int32),
          pltpu.VMEM((_CPB, _CHUNK), jnp.int32),
          pltpu.VMEM((_CHUNK, d), jnp.float32),
          pltpu.VMEM((_CHUNK, d), jnp.float32),
          pltpu.VMEM_SHARED((n_pad, d), jnp.float32),
          pltpu.SemaphoreType.DMA,
          pltpu.SemaphoreType.DMA,
      ],
  )
  def agg_kernel(y_hbm, src_hbm, dst_hbm, out_hbm, src_i, dst_i, msg0, msg1,
                 acc_sh, sem0, sem1):
    cid = lax.axis_index("c")
    sid = lax.axis_index("s")
    wid = sid * _NC + cid
    rbase = sid * rows
    msgs = (msg0, msg1)
    sems = (sem0, sem1)

    _fill_vmem(msg0, _CHUNK, d, 0.0)
    _zero_shared_rows(msg0, acc_sh, rbase, rows)
    plsc.subcore_barrier()

    bbase = wid * blocks_per_tile

    def outer(t, c):
      crow = (bbase + t) * _CPB
      pltpu.sync_copy(src_hbm.at[pl.ds(crow, _CPB)], src_i)
      pltpu.sync_copy(dst_hbm.at[pl.ds(crow, _CPB)], dst_i)
      desc = [None, None]
      desc[0] = pltpu.async_copy(y_hbm.at[src_i.at[0]], msgs[0], sems[0])
      for j in range(_CPB):
        b = j % 2
        if j + 1 < _CPB:
          desc[1 - b] = pltpu.async_copy(
              y_hbm.at[src_i.at[j + 1]], msgs[1 - b], sems[1 - b]
          )
        desc[b].wait()
        pltpu.sync_copy(msgs[b], acc_sh.at[dst_i.at[j]], add=True)
      return c

    lax.fori_loop(0, blocks_per_tile, outer, 0)
    plsc.subcore_barrier()
    pltpu.sync_copy(
        acc_sh.at[pl.ds(rbase, rows)], out_hbm.at[cid, pl.ds(rbase, rows)]
    )

  return agg_kernel


def _tc_prep(X, W1, degp, blk=1000):
  n, d_in = X.shape
  d_hid = W1.shape[1]

  def body(x_ref, w_ref, d0_ref, d1_ref, y_ref, dinv_ref):
    deg = d0_ref[0][:, 0:1] + d1_ref[0][:, 0:1] + 1.0
    dinv = lax.rsqrt(deg)
    xw = jnp.dot(x_ref[...], w_ref[...], preferred_element_type=jnp.float32)
    y_ref[...] = xw * dinv
    dinv_ref[...] = dinv

  return pl.pallas_call(
      body,
      grid=(n // blk,),
      in_specs=[
          pl.BlockSpec((blk, d_in), lambda i: (i, 0)),
          pl.BlockSpec((d_in, d_hid), lambda i: (0, 0)),
          pl.BlockSpec((1, blk, _L), lambda i: (0, i, 0)),
          pl.BlockSpec((1, blk, _L), lambda i: (1, i, 0)),
      ],
      out_specs=[
          pl.BlockSpec((blk, d_hid), lambda i: (i, 0)),
          pl.BlockSpec((blk, 1), lambda i: (i, 0)),
      ],
      out_shape=[
          jax.ShapeDtypeStruct((n, d_hid), jnp.float32),
          jax.ShapeDtypeStruct((n, 1), jnp.float32),
      ],
  )(X, W1, degp, degp)


def _tc_mid(aggp, y, dinv, b1, W2, blk=1000):
  n, d_hid = y.shape
  d_out = W2.shape[1]

  def body(p0_ref, p1_ref, y_ref, dinv_ref, b1_ref, w2_ref, y2_ref):
    dinv = dinv_ref[...]
    pre = dinv * (p0_ref[0] + p1_ref[0] + y_ref[...]) + b1_ref[...]
    h = jnp.maximum(pre, 0.0)
    z = jnp.dot(h, w2_ref[...], preferred_element_type=jnp.float32)
    y2_ref[...] = z * dinv

  return pl.pallas_call(
      body,
      grid=(n // blk,),
      in_specs=[
          pl.BlockSpec((1, blk, d_hid), lambda i: (0, i, 0)),
          pl.BlockSpec((1, blk, d_hid), lambda i: (1, i, 0)),
          pl.BlockSpec((blk, d_hid), lambda i: (i, 0)),
          pl.BlockSpec((blk, 1), lambda i: (i, 0)),
          pl.BlockSpec((1, d_hid), lambda i: (0, 0)),
          pl.BlockSpec((d_hid, d_out), lambda i: (0, 0)),
      ],
      out_specs=pl.BlockSpec((blk, d_out), lambda i: (i, 0)),
      out_shape=jax.ShapeDtypeStruct((n, d_out), jnp.float32),
  )(aggp, aggp, y, dinv, b1.reshape(1, -1), W2)


def _tc_final(aggp, y2, dinv, b2, blk=1000):
  n, d_out = y2.shape

  def body(q0_ref, q1_ref, y2_ref, dinv_ref, b2_ref, out_ref):
    o = dinv_ref[...] * (q0_ref[0] + q1_ref[0] + y2_ref[...]) + b2_ref[...]
    m = jnp.max(o, axis=1, keepdims=True)
    e = jnp.exp(o - m)
    s = jnp.sum(e, axis=1, keepdims=True)
    out_ref[...] = (o - m) - jnp.log(s)

  return pl.pallas_call(
      body,
      grid=(n // blk,),
      in_specs=[
          pl.BlockSpec((1, blk, d_out), lambda i: (0, i, 0)),
          pl.BlockSpec((1, blk, d_out), lambda i: (1, i, 0)),
          pl.BlockSpec((blk, d_out), lambda i: (i, 0)),
          pl.BlockSpec((blk, 1), lambda i: (i, 0)),
          pl.BlockSpec((1, d_out), lambda i: (0, 0)),
      ],
      out_specs=pl.BlockSpec((blk, d_out), lambda i: (i, 0)),
      out_shape=jax.ShapeDtypeStruct((n, d_out), jnp.float32),
  )(aggp, aggp, y2, dinv, b2.reshape(1, -1))


def kernel(X, edge_index, W1, b1, W2, b2):
  n, _ = X.shape
  e = edge_index.shape[1]

  src = edge_index[0].astype(jnp.int32)
  dst = edge_index[1].astype(jnp.int32)

  blk_edges = _NW * _CHUNK * _CPB
  blocks_per_tile = -(-e // blk_edges)
  e_pad = blocks_per_tile * blk_edges
  n_pad = ((n + 1 + _NS * 8 - 1) // (_NS * 8)) * (_NS * 8)
  if e_pad > e:
    src = jnp.concatenate([src, jnp.zeros((e_pad - e,), jnp.int32)])
    dst = jnp.concatenate([dst, jnp.full((e_pad - e,), n, jnp.int32)])
  src2d = src.reshape(e_pad // _CHUNK, _CHUNK)
  dst2d = dst.reshape(e_pad // _CHUNK, _CHUNK)

  degp = _make_deg_kernel(n_pad, blocks_per_tile)(dst2d)
  y1, dinv = _tc_prep(X, W1, degp)
  agg1 = _make_agg_kernel(n_pad, W1.shape[1], blocks_per_tile)(y1, src2d, dst2d)
  y2 = _tc_mid(agg1, y1, dinv, b1, W2)
  agg2 = _make_agg_kernel(n_pad, W2.shape[1], blocks_per_tile)(y2, src2d, dst2d)
  return _tc_final(agg2, y2, dinv, b2)


# speedup vs baseline: 1.2731x; 1.2731x over previous
import functools

import jax
import jax.numpy as jnp
from jax import lax
from jax.experimental import pallas as pl
from jax.experimental.pallas import tpu as pltpu
from jax.experimental.pallas import tpu_sc as plsc

_NC = 2
_NS = 16
_NW = _NC * _NS
_L = 16
_CHUNK = 128
_CPB = 8
_DEPTH = 4
_LEAD = 2


def _fill_vmem(ref, rows, cols, value):
  vec = jnp.full((_L,), value, jnp.float32)

  def body(i, c):
    for j in range(cols // _L):
      ref[i, pl.ds(j * _L, _L)] = vec
    return c

  lax.fori_loop(0, rows, body, 0)


def _zero_shared_rows(src_v, acc_sh, rbase, rows):
  nfull, rem = divmod(rows, _CHUNK)
  for k in range(nfull):
    pltpu.sync_copy(src_v, acc_sh.at[pl.ds(rbase + k * _CHUNK, _CHUNK)])
  if rem:
    pltpu.sync_copy(
        src_v.at[pl.ds(0, rem)],
        acc_sh.at[pl.ds(rbase + nfull * _CHUNK, rem)],
    )


def _make_deg_kernel(n_pad, blocks_per_tile):
  rows = n_pad // _NS

  @functools.partial(
      pl.kernel,
      out_type=jax.ShapeDtypeStruct((_NC, n_pad, _L), jnp.float32),
      mesh=plsc.VectorSubcoreMesh(core_axis_name="c", subcore_axis_name="s"),
      compiler_params=pltpu.CompilerParams(use_tc_tiling_on_sc=False),
      scratch_types=[
          pltpu.VMEM((_CPB, _CHUNK), jnp.int32),
          pltpu.VMEM((_CHUNK, _L), jnp.float32),
          pltpu.VMEM_SHARED((n_pad, _L), jnp.float32),
      ],
  )
  def deg_kernel(dst_hbm, out_hbm, dst_i, val_v, acc_sh):
    cid = lax.axis_index("c")
    sid = lax.axis_index("s")
    wid = sid * _NC + cid
    rbase = sid * rows

    _fill_vmem(val_v, _CHUNK, _L, 0.0)
    _zero_shared_rows(val_v, acc_sh, rbase, rows)
    plsc.subcore_barrier()

    _fill_vmem(val_v, _CHUNK, _L, 1.0)
    bbase = wid * blocks_per_tile

    def outer(t, c):
      crow = (bbase + t) * _CPB
      pltpu.sync_copy(dst_hbm.at[pl.ds(crow, _CPB)], dst_i)
      for j in range(_CPB):
        pltpu.sync_copy(val_v, acc_sh.at[dst_i.at[j]], add=True)
      return c

    lax.fori_loop(0, blocks_per_tile, outer, 0)
    plsc.subcore_barrier()
    pltpu.sync_copy(
        acc_sh.at[pl.ds(rbase, rows)], out_hbm.at[cid, pl.ds(rbase, rows)]
    )

  return deg_kernel


def _make_agg_kernel(n_pad, dh, blocks_per_tile):
  rows = n_pad // _NS
  assert blocks_per_tile % 2 == 0
  body_chunks = 2 * _CPB

  @functools.partial(
      pl.kernel,
      out_type=jax.ShapeDtypeStruct((_NC, n_pad, dh), jnp.float32),
      mesh=plsc.VectorSubcoreMesh(core_axis_name="c", subcore_axis_name="s"),
      compiler_params=pltpu.CompilerParams(use_tc_tiling_on_sc=False),
      scratch_types=[
          [pltpu.VMEM((_CPB, _CHUNK), jnp.int32) for _ in range(2)],
          [pltpu.VMEM((_CPB, _CHUNK), jnp.int32) for _ in range(2)],
          [pltpu.VMEM((_CHUNK, dh), jnp.float32) for _ in range(_DEPTH)],
          pltpu.VMEM_SHARED((n_pad, dh), jnp.float32),
          [pltpu.SemaphoreType.DMA for _ in range(_DEPTH)],
          [pltpu.SemaphoreType.DMA for _ in range(_DEPTH)],
          [pltpu.SemaphoreType.DMA for _ in range(4)],
      ],
  )
  def agg_kernel(y_hbm, src_hbm, dst_hbm, out_hbm, src_i, dst_i, msgs,
                 acc_sh, sem_g, sem_s, sem_i):
    cid = lax.axis_index("c")
    sid = lax.axis_index("s")
    rbase = sid * rows

    _fill_vmem(msgs[0], _CHUNK, dh, 0.0)
    _zero_shared_rows(msgs[0], acc_sh, rbase, rows)
    plsc.subcore_barrier()

    tile_rows = blocks_per_tile * _CPB
    src_base = (cid * _NS + sid) * tile_rows
    dst_base = sid * tile_rows

    def body(k, carry):
      srow = src_base + 2 * k * _CPB
      drow = dst_base + 2 * k * _CPB
      idesc = [
          pltpu.async_copy(src_hbm.at[pl.ds(srow, _CPB)], src_i[0], sem_i[0]),
          pltpu.async_copy(dst_hbm.at[pl.ds(drow, _CPB)], dst_i[0], sem_i[1]),
          pltpu.async_copy(src_hbm.at[pl.ds(srow + _CPB, _CPB)], src_i[1],
                           sem_i[2]),
          pltpu.async_copy(dst_hbm.at[pl.ds(drow + _CPB, _CPB)], dst_i[1],
                           sem_i[3]),
      ]
      gdesc = [None] * _DEPTH
      sdesc = [None] * _DEPTH
      for it in range(body_chunks + _LEAD):
        c = it - _LEAD
        if c >= 0:
          b = c % _DEPTH
          gdesc[b].wait()
          p, jc = divmod(c, _CPB)
          sdesc[b] = pltpu.async_copy(
              msgs[b], acc_sh.at[dst_i[p].at[jc]], sem_s[b], add=True
          )
        if it < body_chunks:
          p, j = divmod(it, _CPB)
          if j == 0:
            idesc[2 * p].wait()
            idesc[2 * p + 1].wait()
          b2 = it % _DEPTH
          if it >= _DEPTH:
            sdesc[b2].wait()
          gdesc[b2] = pltpu.async_copy(
              y_hbm.at[src_i[p].at[j]], msgs[b2], sem_g[b2]
          )
      for b in range(_DEPTH):
        sdesc[b].wait()
      return carry

    lax.fori_loop(0, blocks_per_tile // 2, body, 0)
    plsc.subcore_barrier()
    pltpu.sync_copy(
        acc_sh.at[pl.ds(rbase, rows)], out_hbm.at[cid, pl.ds(rbase, rows)]
    )

  return agg_kernel


def _tc_prep(X, W1, degp, blk=1000):
  n, d_in = X.shape
  d_hid = W1.shape[1]
  dh = d_hid // 2

  def body(x_ref, w_ref, d0_ref, d1_ref, y_ref, dinv_ref):
    deg = d0_ref[0][:, 0:1] + d1_ref[0][:, 0:1] + 1.0
    dinv = lax.rsqrt(deg)
    xw = jnp.dot(x_ref[...], w_ref[...], preferred_element_type=jnp.float32)
    y = xw * dinv
    y_ref[0] = y[:, :dh]
    y_ref[1] = y[:, dh:]
    dinv_ref[...] = dinv

  return pl.pallas_call(
      body,
      grid=(n // blk,),
      in_specs=[
          pl.BlockSpec((blk, d_in), lambda i: (i, 0)),
          pl.BlockSpec((d_in, d_hid), lambda i: (0, 0)),
          pl.BlockSpec((1, blk, _L), lambda i: (0, i, 0)),
          pl.BlockSpec((1, blk, _L), lambda i: (1, i, 0)),
      ],
      out_specs=[
          pl.BlockSpec((2, blk, dh), lambda i: (0, i, 0)),
          pl.BlockSpec((blk, 1), lambda i: (i, 0)),
      ],
      out_shape=[
          jax.ShapeDtypeStruct((2, n, dh), jnp.float32),
          jax.ShapeDtypeStruct((n, 1), jnp.float32),
      ],
  )(X, W1, degp, degp)


def _tc_mid(aggp, y, dinv, b1, W2, blk=1000):
  n = dinv.shape[0]
  d_out = W2.shape[1]
  dh2 = d_out // 2

  def body(p_ref, y_ref, dinv_ref, b1_ref, w2_ref, y2_ref):
    dinv = dinv_ref[...]
    agg = jnp.concatenate([p_ref[0], p_ref[1]], axis=1)
    yy = jnp.concatenate([y_ref[0], y_ref[1]], axis=1)
    pre = dinv * (agg + yy) + b1_ref[...]
    h = jnp.maximum(pre, 0.0)
    z = jnp.dot(h, w2_ref[...], preferred_element_type=jnp.float32)
    zz = z * dinv
    y2_ref[0] = zz[:, :dh2]
    y2_ref[1] = zz[:, dh2:]

  d_hid = 2 * y.shape[2]
  return pl.pallas_call(
      body,
      grid=(n // blk,),
      in_specs=[
          pl.BlockSpec((2, blk, d_hid // 2), lambda i: (0, i, 0)),
          pl.BlockSpec((2, blk, d_hid // 2), lambda i: (0, i, 0)),
          pl.BlockSpec((blk, 1), lambda i: (i, 0)),
          pl.BlockSpec((1, d_hid), lambda i: (0, 0)),
          pl.BlockSpec((d_hid, d_out), lambda i: (0, 0)),
      ],
      out_specs=pl.BlockSpec((2, blk, dh2), lambda i: (0, i, 0)),
      out_shape=jax.ShapeDtypeStruct((2, n, dh2), jnp.float32),
  )(aggp, y, dinv, b1.reshape(1, -1), W2)


def _tc_final(aggp, y2, dinv, b2, blk=1000):
  n = dinv.shape[0]
  dh2 = y2.shape[2]
  d_out = 2 * dh2

  def body(q_ref, y2_ref, dinv_ref, b2_ref, out_ref):
    agg = jnp.concatenate([q_ref[0], q_ref[1]], axis=1)
    yy = jnp.concatenate([y2_ref[0], y2_ref[1]], axis=1)
    o = dinv_ref[...] * (agg + yy) + b2_ref[...]
    m = jnp.max(o, axis=1, keepdims=True)
    e = jnp.exp(o - m)
    s = jnp.sum(e, axis=1, keepdims=True)
    out_ref[...] = (o - m) - jnp.log(s)

  return pl.pallas_call(
      body,
      grid=(n // blk,),
      in_specs=[
          pl.BlockSpec((2, blk, dh2), lambda i: (0, i, 0)),
          pl.BlockSpec((2, blk, dh2), lambda i: (0, i, 0)),
          pl.BlockSpec((blk, 1), lambda i: (i, 0)),
          pl.BlockSpec((1, d_out), lambda i: (0, 0)),
      ],
      out_specs=pl.BlockSpec((blk, d_out), lambda i: (i, 0)),
      out_shape=jax.ShapeDtypeStruct((n, d_out), jnp.float32),
  )(aggp, y2, dinv, b2.reshape(1, -1))


def kernel(X, edge_index, W1, b1, W2, b2):
  n, _ = X.shape
  e = edge_index.shape[1]

  src = edge_index[0].astype(jnp.int32)
  dst = edge_index[1].astype(jnp.int32)

  blk_edges = _NS * _CHUNK * _CPB * 2
  e_pad = -(-e // blk_edges) * blk_edges
  agg_bpt = e_pad // (_NS * _CHUNK * _CPB)
  deg_bpt = e_pad // (_NW * _CHUNK * _CPB)
  n_pad = ((n + 1 + _NS * 8 - 1) // (_NS * 8)) * (_NS * 8)
  if e_pad > e:
    src = jnp.concatenate([src, jnp.zeros((e_pad - e,), jnp.int32)])
    dst = jnp.concatenate([dst, jnp.full((e_pad - e,), n, jnp.int32)])
  nrows = e_pad // _CHUNK
  src2d = jnp.stack([src, src + n]).reshape(2 * nrows, _CHUNK)
  dst2d = dst.reshape(nrows, _CHUNK)

  degp = _make_deg_kernel(n_pad, deg_bpt)(dst2d)
  y1, dinv = _tc_prep(X, W1, degp)
  d_hid = W1.shape[1]
  d_out = W2.shape[1]
  agg1 = _make_agg_kernel(n_pad, d_hid // 2, agg_bpt)(
      y1.reshape(2 * n, d_hid // 2), src2d, dst2d)
  y2 = _tc_mid(agg1, y1, dinv, b1, W2)
  agg2 = _make_agg_kernel(n_pad, d_out // 2, agg_bpt)(
      y2.reshape(2 * n, d_out // 2), src2d, dst2d)
  return _tc_final(agg2, y2, dinv, b2)


# speedup vs baseline: 1.3011x; 1.0220x over previous
import functools

import jax
import jax.numpy as jnp
from jax import lax
from jax.experimental import pallas as pl
from jax.experimental.pallas import tpu as pltpu
from jax.experimental.pallas import tpu_sc as plsc

_NC = 2
_NS = 16
_NW = _NC * _NS
_L = 16
_CHUNK = 128
_CPB = 8
_DEPTH = 8
_LEAD = 4


def _fill_vmem(ref, rows, cols, value):
  vec = jnp.full((_L,), value, jnp.float32)

  def body(i, c):
    for j in range(cols // _L):
      ref[i, pl.ds(j * _L, _L)] = vec
    return c

  lax.fori_loop(0, rows, body, 0)


def _zero_shared_rows(src_v, acc_sh, rbase, rows):
  nfull, rem = divmod(rows, _CHUNK)
  for k in range(nfull):
    pltpu.sync_copy(src_v, acc_sh.at[pl.ds(rbase + k * _CHUNK, _CHUNK)])
  if rem:
    pltpu.sync_copy(
        src_v.at[pl.ds(0, rem)],
        acc_sh.at[pl.ds(rbase + nfull * _CHUNK, rem)],
    )


def _make_deg_kernel(n_pad, blocks_per_tile):
  rows = n_pad // _NS

  @functools.partial(
      pl.kernel,
      out_type=jax.ShapeDtypeStruct((_NC, n_pad, _L), jnp.float32),
      mesh=plsc.VectorSubcoreMesh(core_axis_name="c", subcore_axis_name="s"),
      compiler_params=pltpu.CompilerParams(use_tc_tiling_on_sc=False),
      scratch_types=[
          pltpu.VMEM((_CPB, _CHUNK), jnp.int32),
          pltpu.VMEM((_CHUNK, _L), jnp.float32),
          pltpu.VMEM_SHARED((n_pad, _L), jnp.float32),
      ],
  )
  def deg_kernel(dst_hbm, out_hbm, dst_i, val_v, acc_sh):
    cid = lax.axis_index("c")
    sid = lax.axis_index("s")
    wid = sid * _NC + cid
    rbase = sid * rows

    _fill_vmem(val_v, _CHUNK, _L, 0.0)
    _zero_shared_rows(val_v, acc_sh, rbase, rows)
    plsc.subcore_barrier()

    _fill_vmem(val_v, _CHUNK, _L, 1.0)
    bbase = wid * blocks_per_tile

    def outer(t, c):
      crow = (bbase + t) * _CPB
      pltpu.sync_copy(dst_hbm.at[pl.ds(crow, _CPB)], dst_i)
      for j in range(_CPB):
        pltpu.sync_copy(val_v, acc_sh.at[dst_i.at[j]], add=True)
      return c

    lax.fori_loop(0, blocks_per_tile, outer, 0)
    plsc.subcore_barrier()
    pltpu.sync_copy(
        acc_sh.at[pl.ds(rbase, rows)], out_hbm.at[cid, pl.ds(rbase, rows)]
    )

  return deg_kernel


def _make_agg_kernel(n_pad, dh, blocks_per_tile):
  rows = n_pad // _NS
  assert blocks_per_tile % 2 == 0
  body_chunks = 2 * _CPB

  @functools.partial(
      pl.kernel,
      out_type=jax.ShapeDtypeStruct((_NC, n_pad, dh), jnp.float32),
      mesh=plsc.VectorSubcoreMesh(core_axis_name="c", subcore_axis_name="s"),
      compiler_params=pltpu.CompilerParams(use_tc_tiling_on_sc=False),
      scratch_types=[
          [pltpu.VMEM((_CPB, _CHUNK), jnp.int32) for _ in range(2)],
          [pltpu.VMEM((_CPB, _CHUNK), jnp.int32) for _ in range(2)],
          [pltpu.VMEM((_CHUNK, dh), jnp.float32) for _ in range(_DEPTH)],
          pltpu.VMEM_SHARED((n_pad, dh), jnp.float32),
          [pltpu.SemaphoreType.DMA for _ in range(_DEPTH)],
          [pltpu.SemaphoreType.DMA for _ in range(_DEPTH)],
          [pltpu.SemaphoreType.DMA for _ in range(4)],
      ],
  )
  def agg_kernel(y_hbm, src_hbm, dst_hbm, out_hbm, src_i, dst_i, msgs,
                 acc_sh, sem_g, sem_s, sem_i):
    cid = lax.axis_index("c")
    sid = lax.axis_index("s")
    rbase = sid * rows

    _fill_vmem(msgs[0], _CHUNK, dh, 0.0)
    _zero_shared_rows(msgs[0], acc_sh, rbase, rows)
    plsc.subcore_barrier()

    tile_rows = blocks_per_tile * _CPB
    src_base = (cid * _NS + sid) * tile_rows
    dst_base = sid * tile_rows

    def body(k, carry):
      srow = src_base + 2 * k * _CPB
      drow = dst_base + 2 * k * _CPB
      idesc = [
          pltpu.async_copy(src_hbm.at[pl.ds(srow, _CPB)], src_i[0], sem_i[0]),
          pltpu.async_copy(dst_hbm.at[pl.ds(drow, _CPB)], dst_i[0], sem_i[1]),
          pltpu.async_copy(src_hbm.at[pl.ds(srow + _CPB, _CPB)], src_i[1],
                           sem_i[2]),
          pltpu.async_copy(dst_hbm.at[pl.ds(drow + _CPB, _CPB)], dst_i[1],
                           sem_i[3]),
      ]
      gdesc = [None] * _DEPTH
      sdesc = [None] * _DEPTH
      for it in range(body_chunks + _LEAD):
        c = it - _LEAD
        if c >= 0:
          b = c % _DEPTH
          gdesc[b].wait()
          p, jc = divmod(c, _CPB)
          sdesc[b] = pltpu.async_copy(
              msgs[b], acc_sh.at[dst_i[p].at[jc]], sem_s[b], add=True
          )
        if it < body_chunks:
          p, j = divmod(it, _CPB)
          if j == 0:
            idesc[2 * p].wait()
            idesc[2 * p + 1].wait()
          b2 = it % _DEPTH
          if it >= _DEPTH:
            sdesc[b2].wait()
          gdesc[b2] = pltpu.async_copy(
              y_hbm.at[src_i[p].at[j]], msgs[b2], sem_g[b2]
          )
      for b in range(_DEPTH):
        sdesc[b].wait()
      return carry

    lax.fori_loop(0, blocks_per_tile // 2, body, 0)
    plsc.subcore_barrier()
    pltpu.sync_copy(
        acc_sh.at[pl.ds(rbase, rows)], out_hbm.at[cid, pl.ds(rbase, rows)]
    )

  return agg_kernel


def _tc_prep(X, W1, degp, blk=1000):
  n, d_in = X.shape
  d_hid = W1.shape[1]
  dh = d_hid // 2

  def body(x_ref, w_ref, d0_ref, d1_ref, y_ref, dinv_ref):
    deg = d0_ref[0][:, 0:1] + d1_ref[0][:, 0:1] + 1.0
    dinv = lax.rsqrt(deg)
    xw = jnp.dot(x_ref[...], w_ref[...], preferred_element_type=jnp.float32)
    y = xw * dinv
    y_ref[0] = y[:, :dh]
    y_ref[1] = y[:, dh:]
    dinv_ref[...] = dinv

  return pl.pallas_call(
      body,
      grid=(n // blk,),
      in_specs=[
          pl.BlockSpec((blk, d_in), lambda i: (i, 0)),
          pl.BlockSpec((d_in, d_hid), lambda i: (0, 0)),
          pl.BlockSpec((1, blk, _L), lambda i: (0, i, 0)),
          pl.BlockSpec((1, blk, _L), lambda i: (1, i, 0)),
      ],
      out_specs=[
          pl.BlockSpec((2, blk, dh), lambda i: (0, i, 0)),
          pl.BlockSpec((blk, 1), lambda i: (i, 0)),
      ],
      out_shape=[
          jax.ShapeDtypeStruct((2, n, dh), jnp.float32),
          jax.ShapeDtypeStruct((n, 1), jnp.float32),
      ],
  )(X, W1, degp, degp)


def _tc_mid(aggp, y, dinv, b1, W2, blk=1000):
  n = dinv.shape[0]
  d_out = W2.shape[1]
  dh2 = d_out // 2

  def body(p_ref, y_ref, dinv_ref, b1_ref, w2_ref, y2_ref):
    dinv = dinv_ref[...]
    agg = jnp.concatenate([p_ref[0], p_ref[1]], axis=1)
    yy = jnp.concatenate([y_ref[0], y_ref[1]], axis=1)
    pre = dinv * (agg + yy) + b1_ref[...]
    h = jnp.maximum(pre, 0.0)
    z = jnp.dot(h, w2_ref[...], preferred_element_type=jnp.float32)
    zz = z * dinv
    y2_ref[0] = zz[:, :dh2]
    y2_ref[1] = zz[:, dh2:]

  d_hid = 2 * y.shape[2]
  return pl.pallas_call(
      body,
      grid=(n // blk,),
      in_specs=[
          pl.BlockSpec((2, blk, d_hid // 2), lambda i: (0, i, 0)),
          pl.BlockSpec((2, blk, d_hid // 2), lambda i: (0, i, 0)),
          pl.BlockSpec((blk, 1), lambda i: (i, 0)),
          pl.BlockSpec((1, d_hid), lambda i: (0, 0)),
          pl.BlockSpec((d_hid, d_out), lambda i: (0, 0)),
      ],
      out_specs=pl.BlockSpec((2, blk, dh2), lambda i: (0, i, 0)),
      out_shape=jax.ShapeDtypeStruct((2, n, dh2), jnp.float32),
  )(aggp, y, dinv, b1.reshape(1, -1), W2)


def _tc_final(aggp, y2, dinv, b2, blk=1000):
  n = dinv.shape[0]
  dh2 = y2.shape[2]
  d_out = 2 * dh2

  def body(q_ref, y2_ref, dinv_ref, b2_ref, out_ref):
    agg = jnp.concatenate([q_ref[0], q_ref[1]], axis=1)
    yy = jnp.concatenate([y2_ref[0], y2_ref[1]], axis=1)
    o = dinv_ref[...] * (agg + yy) + b2_ref[...]
    m = jnp.max(o, axis=1, keepdims=True)
    e = jnp.exp(o - m)
    s = jnp.sum(e, axis=1, keepdims=True)
    out_ref[...] = (o - m) - jnp.log(s)

  return pl.pallas_call(
      body,
      grid=(n // blk,),
      in_specs=[
          pl.BlockSpec((2, blk, dh2), lambda i: (0, i, 0)),
          pl.BlockSpec((2, blk, dh2), lambda i: (0, i, 0)),
          pl.BlockSpec((blk, 1), lambda i: (i, 0)),
          pl.BlockSpec((1, d_out), lambda i: (0, 0)),
      ],
      out_specs=pl.BlockSpec((blk, d_out), lambda i: (i, 0)),
      out_shape=jax.ShapeDtypeStruct((n, d_out), jnp.float32),
  )(aggp, y2, dinv, b2.reshape(1, -1))


def kernel(X, edge_index, W1, b1, W2, b2):
  n, _ = X.shape
  e = edge_index.shape[1]

  src = edge_index[0].astype(jnp.int32)
  dst = edge_index[1].astype(jnp.int32)

  blk_edges = _NS * _CHUNK * _CPB * 2
  e_pad = -(-e // blk_edges) * blk_edges
  agg_bpt = e_pad // (_NS * _CHUNK * _CPB)
  deg_bpt = e_pad // (_NW * _CHUNK * _CPB)
  n_pad = ((n + 1 + _NS * 8 - 1) // (_NS * 8)) * (_NS * 8)
  if e_pad > e:
    src = jnp.concatenate([src, jnp.zeros((e_pad - e,), jnp.int32)])
    dst = jnp.concatenate([dst, jnp.full((e_pad - e,), n, jnp.int32)])
  nrows = e_pad // _CHUNK
  src2d = jnp.stack([src, src + n]).reshape(2 * nrows, _CHUNK)
  dst2d = dst.reshape(nrows, _CHUNK)

  degp = _make_deg_kernel(n_pad, deg_bpt)(dst2d)
  y1, dinv = _tc_prep(X, W1, degp)
  d_hid = W1.shape[1]
  d_out = W2.shape[1]
  agg1 = _make_agg_kernel(n_pad, d_hid // 2, agg_bpt)(
      y1.reshape(2 * n, d_hid // 2), src2d, dst2d)
  y2 = _tc_mid(agg1, y1, dinv, b1, W2)
  agg2 = _make_agg_kernel(n_pad, d_out // 2, agg_bpt)(
      y2.reshape(2 * n, d_out // 2), src2d, dst2d)
  return _tc_final(agg2, y2, dinv, b2)


# speedup vs baseline: 2.5387x; 1.9511x over previous
import functools

import jax
import jax.numpy as jnp
from jax import lax
from jax.experimental import pallas as pl
from jax.experimental.pallas import tpu as pltpu
from jax.experimental.pallas import tpu_sc as plsc

_NC = 2
_NS = 16
_NW = _NC * _NS
_L = 16
_CHUNK = 128
_CPB = 8


def _fill_vmem(ref, rows, cols, value):
  vec = jnp.full((_L,), value, jnp.float32)

  def body(i, c):
    for j in range(cols // _L):
      ref[i, pl.ds(j * _L, _L)] = vec
    return c

  lax.fori_loop(0, rows, body, 0)


def _zero_shared_rows(src_v, acc_sh, rbase, rows):
  nfull, rem = divmod(rows, _CHUNK)
  for k in range(nfull):
    pltpu.sync_copy(src_v, acc_sh.at[pl.ds(rbase + k * _CHUNK, _CHUNK)])
  if rem:
    pltpu.sync_copy(
        src_v.at[pl.ds(0, rem)],
        acc_sh.at[pl.ds(rbase + nfull * _CHUNK, rem)],
    )


def _make_deg_kernel(n_pad, blocks_per_tile):
  rows = n_pad // _NS

  @functools.partial(
      pl.kernel,
      out_type=jax.ShapeDtypeStruct((_NC, n_pad, _L), jnp.float32),
      mesh=plsc.VectorSubcoreMesh(core_axis_name="c", subcore_axis_name="s"),
      compiler_params=pltpu.CompilerParams(use_tc_tiling_on_sc=False),
      scratch_types=[
          pltpu.VMEM((_CPB, _CHUNK), jnp.int32),
          pltpu.VMEM((_CHUNK, _L), jnp.float32),
          pltpu.VMEM_SHARED((n_pad, _L), jnp.float32),
      ],
  )
  def deg_kernel(dst_hbm, out_hbm, dst_i, val_v, acc_sh):
    cid = lax.axis_index("c")
    sid = lax.axis_index("s")
    wid = sid * _NC + cid
    rbase = sid * rows

    _fill_vmem(val_v, _CHUNK, _L, 0.0)
    _zero_shared_rows(val_v, acc_sh, rbase, rows)
    plsc.subcore_barrier()

    _fill_vmem(val_v, _CHUNK, _L, 1.0)
    bbase = wid * blocks_per_tile

    def outer(t, c):
      crow = (bbase + t) * _CPB
      pltpu.sync_copy(dst_hbm.at[pl.ds(crow, _CPB)], dst_i)
      for j in range(_CPB):
        pltpu.sync_copy(val_v, acc_sh.at[dst_i.at[j]], add=True)
      return c

    lax.fori_loop(0, blocks_per_tile, outer, 0)
    plsc.subcore_barrier()
    pltpu.sync_copy(
        acc_sh.at[pl.ds(rbase, rows)], out_hbm.at[cid, pl.ds(rbase, rows)]
    )

  return deg_kernel


def _make_agg_kernel(n_pad, d, blocks_per_tile, depth, lead):
  rows = n_pad // _NS
  assert blocks_per_tile % 2 == 0
  assert lead % depth != 0 and (depth - lead) % depth != 0
  body_chunks = 2 * _CPB

  @functools.partial(
      pl.kernel,
      out_type=jax.ShapeDtypeStruct((_NC, n_pad, d), jnp.float32),
      mesh=plsc.VectorSubcoreMesh(core_axis_name="c", subcore_axis_name="s"),
      compiler_params=pltpu.CompilerParams(use_tc_tiling_on_sc=False),
      scratch_types=[
          [pltpu.VMEM((_CPB, _CHUNK), jnp.int32) for _ in range(2)],
          [pltpu.VMEM((_CPB, _CHUNK), jnp.int32) for _ in range(2)],
          [pltpu.VMEM((_CHUNK, d), jnp.float32) for _ in range(depth)],
          pltpu.VMEM_SHARED((n_pad, d), jnp.float32),
          [pltpu.SemaphoreType.DMA for _ in range(depth)],
          [pltpu.SemaphoreType.DMA for _ in range(depth)],
          [pltpu.SemaphoreType.DMA for _ in range(4)],
      ],
  )
  def agg_kernel(y_hbm, src_hbm, dst_hbm, out_hbm, src_i, dst_i, msgs,
                 acc_sh, sem_g, sem_s, sem_i):
    cid = lax.axis_index("c")
    sid = lax.axis_index("s")
    wid = sid * _NC + cid
    rbase = sid * rows

    _fill_vmem(msgs[0], _CHUNK, d, 0.0)
    _zero_shared_rows(msgs[0], acc_sh, rbase, rows)
    plsc.subcore_barrier()

    base = wid * blocks_per_tile * _CPB

    def body(k, carry):
      crow = base + 2 * k * _CPB
      idesc = [
          pltpu.async_copy(src_hbm.at[pl.ds(crow, _CPB)], src_i[0], sem_i[0]),
          pltpu.async_copy(dst_hbm.at[pl.ds(crow, _CPB)], dst_i[0], sem_i[1]),
          pltpu.async_copy(src_hbm.at[pl.ds(crow + _CPB, _CPB)], src_i[1],
                           sem_i[2]),
          pltpu.async_copy(dst_hbm.at[pl.ds(crow + _CPB, _CPB)], dst_i[1],
                           sem_i[3]),
      ]
      gdesc = [None] * depth
      sdesc = [None] * depth
      for it in range(body_chunks + lead):
        c = it - lead
        if c >= 0:
          b = c % depth
          gdesc[b].wait()
          p, jc = divmod(c, _CPB)
          sdesc[b] = pltpu.async_copy(
              msgs[b], acc_sh.at[dst_i[p].at[jc]], sem_s[b], add=True
          )
        if it < body_chunks:
          p, j = divmod(it, _CPB)
          if j == 0:
            idesc[2 * p].wait()
            idesc[2 * p + 1].wait()
          b2 = it % depth
          if it >= depth:
            sdesc[b2].wait()
          gdesc[b2] = pltpu.async_copy(
              y_hbm.at[src_i[p].at[j]], msgs[b2], sem_g[b2]
          )
      for b in range(depth):
        sdesc[b].wait()
      return carry

    lax.fori_loop(0, blocks_per_tile // 2, body, 0)
    plsc.subcore_barrier()
    pltpu.sync_copy(
        acc_sh.at[pl.ds(rbase, rows)], out_hbm.at[cid, pl.ds(rbase, rows)]
    )

  return agg_kernel


def _tc_prep(X, W1, degp, blk=1000):
  n, d_in = X.shape
  d_hid = W1.shape[1]

  def body(x_ref, w_ref, d_ref, y_ref, dinv_ref):
    deg = d_ref[0][:, 0:1] + d_ref[1][:, 0:1] + 1.0
    dinv = lax.rsqrt(deg)
    xw = jnp.dot(x_ref[...], w_ref[...], preferred_element_type=jnp.float32)
    y_ref[...] = xw * dinv
    dinv_ref[...] = dinv

  return pl.pallas_call(
      body,
      grid=(n // blk,),
      in_specs=[
          pl.BlockSpec((blk, d_in), lambda i: (i, 0)),
          pl.BlockSpec((d_in, d_hid), lambda i: (0, 0)),
          pl.BlockSpec((2, blk, _L), lambda i: (0, i, 0)),
      ],
      out_specs=[
          pl.BlockSpec((blk, d_hid), lambda i: (i, 0)),
          pl.BlockSpec((blk, 1), lambda i: (i, 0)),
      ],
      out_shape=[
          jax.ShapeDtypeStruct((n, d_hid), jnp.float32),
          jax.ShapeDtypeStruct((n, 1), jnp.float32),
      ],
  )(X, W1, degp)


def _tc_mid(aggp, y, dinv, b1, W2, blk=1000):
  n, d_hid = y.shape
  d_out = W2.shape[1]

  def body(p_ref, y_ref, dinv_ref, b1_ref, w2_ref, y2_ref):
    dinv = dinv_ref[...]
    pre = dinv * (p_ref[0] + p_ref[1] + y_ref[...]) + b1_ref[...]
    h = jnp.maximum(pre, 0.0)
    z = jnp.dot(h, w2_ref[...], preferred_element_type=jnp.float32)
    y2_ref[...] = z * dinv

  return pl.pallas_call(
      body,
      grid=(n // blk,),
      in_specs=[
          pl.BlockSpec((2, blk, d_hid), lambda i: (0, i, 0)),
          pl.BlockSpec((blk, d_hid), lambda i: (i, 0)),
          pl.BlockSpec((blk, 1), lambda i: (i, 0)),
          pl.BlockSpec((1, d_hid), lambda i: (0, 0)),
          pl.BlockSpec((d_hid, d_out), lambda i: (0, 0)),
      ],
      out_specs=pl.BlockSpec((blk, d_out), lambda i: (i, 0)),
      out_shape=jax.ShapeDtypeStruct((n, d_out), jnp.float32),
  )(aggp, y, dinv, b1.reshape(1, -1), W2)


def _tc_final(aggp, y2, dinv, b2, blk=1000):
  n, d_out = y2.shape

  def body(q_ref, y2_ref, dinv_ref, b2_ref, out_ref):
    o = dinv_ref[...] * (q_ref[0] + q_ref[1] + y2_ref[...]) + b2_ref[...]
    m = jnp.max(o, axis=1, keepdims=True)
    e = jnp.exp(o - m)
    s = jnp.sum(e, axis=1, keepdims=True)
    out_ref[...] = (o - m) - jnp.log(s)

  return pl.pallas_call(
      body,
      grid=(n // blk,),
      in_specs=[
          pl.BlockSpec((2, blk, d_out), lambda i: (0, i, 0)),
          pl.BlockSpec((blk, d_out), lambda i: (i, 0)),
          pl.BlockSpec((blk, 1), lambda i: (i, 0)),
          pl.BlockSpec((1, d_out), lambda i: (0, 0)),
      ],
      out_specs=pl.BlockSpec((blk, d_out), lambda i: (i, 0)),
      out_shape=jax.ShapeDtypeStruct((n, d_out), jnp.float32),
  )(aggp, y2, dinv, b2.reshape(1, -1))


def kernel(X, edge_index, W1, b1, W2, b2):
  n, _ = X.shape
  e = edge_index.shape[1]

  src = edge_index[0].astype(jnp.int32)
  dst = edge_index[1].astype(jnp.int32)

  blk_edges = _NW * _CHUNK * _CPB
  bpt = 2 * -(-e // (2 * blk_edges))
  e_pad = bpt * blk_edges
  n_pad = (n + _NS) // _NS * _NS
  if e_pad > e:
    pad = e_pad - e
    src = jnp.concatenate([src, jnp.arange(pad, dtype=jnp.int32) % n])
    dst = jnp.concatenate(
        [dst, n + jnp.arange(pad, dtype=jnp.int32) % (n_pad - n)])
  nrows = e_pad // _CHUNK
  src2d = src.reshape(nrows, _CHUNK)
  dst2d = dst.reshape(nrows, _CHUNK)

  d_hid = W1.shape[1]
  d_out = W2.shape[1]
  degp = _make_deg_kernel(n_pad, bpt)(dst2d)
  y1, dinv = _tc_prep(X, W1, degp)
  agg1 = _make_agg_kernel(n_pad, d_hid, bpt, 2, 1)(y1, src2d, dst2d)
  y2 = _tc_mid(agg1, y1, dinv, b1, W2)
  agg2 = _make_agg_kernel(n_pad, d_out, bpt, 4, 2)(y2, src2d, dst2d)
  return _tc_final(agg2, y2, dinv, b2)


# speedup vs baseline: 2.5737x; 1.0138x over previous
import functools

import jax
import jax.numpy as jnp
from jax import lax
from jax.experimental import pallas as pl
from jax.experimental.pallas import tpu as pltpu
from jax.experimental.pallas import tpu_sc as plsc

_NC = 2
_NS = 16
_NW = _NC * _NS
_L = 16
_CHUNK = 128
_CPB = 8


def _fill_vmem(ref, rows, cols, value):
  vec = jnp.full((_L,), value, jnp.float32)

  def body(i, c):
    for j in range(cols // _L):
      ref[i, pl.ds(j * _L, _L)] = vec
    return c

  lax.fori_loop(0, rows, body, 0)


def _zero_shared_rows(src_v, acc_sh, rbase, rows):
  nfull, rem = divmod(rows, _CHUNK)
  for k in range(nfull):
    pltpu.sync_copy(src_v, acc_sh.at[pl.ds(rbase + k * _CHUNK, _CHUNK)])
  if rem:
    pltpu.sync_copy(
        src_v.at[pl.ds(0, rem)],
        acc_sh.at[pl.ds(rbase + nfull * _CHUNK, rem)],
    )


def _make_deg_kernel(n_pad, blocks_per_tile):
  rows = n_pad // _NS

  @functools.partial(
      pl.kernel,
      out_type=jax.ShapeDtypeStruct((_NC, n_pad, _L), jnp.float32),
      mesh=plsc.VectorSubcoreMesh(core_axis_name="c", subcore_axis_name="s"),
      compiler_params=pltpu.CompilerParams(use_tc_tiling_on_sc=False),
      scratch_types=[
          pltpu.VMEM((_CPB, _CHUNK), jnp.int32),
          pltpu.VMEM((_CHUNK, _L), jnp.float32),
          pltpu.VMEM_SHARED((n_pad, _L), jnp.float32),
      ],
  )
  def deg_kernel(dst_hbm, out_hbm, dst_i, val_v, acc_sh):
    cid = lax.axis_index("c")
    sid = lax.axis_index("s")
    wid = sid * _NC + cid
    rbase = sid * rows

    _fill_vmem(val_v, _CHUNK, _L, 0.0)
    _zero_shared_rows(val_v, acc_sh, rbase, rows)
    plsc.subcore_barrier()

    _fill_vmem(val_v, _CHUNK, _L, 1.0)
    bbase = wid * blocks_per_tile

    def outer(t, c):
      crow = (bbase + t) * _CPB
      pltpu.sync_copy(dst_hbm.at[pl.ds(crow, _CPB)], dst_i)
      for j in range(_CPB):
        pltpu.sync_copy(val_v, acc_sh.at[dst_i.at[j]], add=True)
      return c

    lax.fori_loop(0, blocks_per_tile, outer, 0)
    plsc.subcore_barrier()
    pltpu.sync_copy(
        acc_sh.at[pl.ds(rbase, rows)], out_hbm.at[cid, pl.ds(rbase, rows)]
    )

  return deg_kernel


def _make_agg_kernel(n_pad, d, blocks_per_tile, depth, lead):
  rows = n_pad // _NS
  assert blocks_per_tile % 2 == 0
  assert lead % depth != 0 and (depth - lead) % depth != 0
  body_chunks = 2 * _CPB

  @functools.partial(
      pl.kernel,
      out_type=jax.ShapeDtypeStruct((_NC, n_pad, d), jnp.float32),
      mesh=plsc.VectorSubcoreMesh(core_axis_name="c", subcore_axis_name="s"),
      compiler_params=pltpu.CompilerParams(use_tc_tiling_on_sc=False),
      scratch_types=[
          [pltpu.VMEM((_CPB, _CHUNK), jnp.int32) for _ in range(2)],
          [pltpu.VMEM((_CPB, _CHUNK), jnp.int32) for _ in range(2)],
          [pltpu.VMEM((_CHUNK, d), jnp.float32) for _ in range(depth)],
          pltpu.VMEM_SHARED((n_pad, d), jnp.float32),
          [pltpu.SemaphoreType.DMA for _ in range(depth)],
          [pltpu.SemaphoreType.DMA for _ in range(depth)],
          [pltpu.SemaphoreType.DMA for _ in range(4)],
      ],
  )
  def agg_kernel(y_hbm, src_hbm, dst_hbm, out_hbm, src_i, dst_i, msgs,
                 acc_sh, sem_g, sem_s, sem_i):
    cid = lax.axis_index("c")
    sid = lax.axis_index("s")
    wid = sid * _NC + cid
    rbase = sid * rows

    _fill_vmem(msgs[0], _CHUNK, d, 0.0)
    _zero_shared_rows(msgs[0], acc_sh, rbase, rows)
    plsc.subcore_barrier()

    base = wid * blocks_per_tile * _CPB

    def body(k, carry):
      crow = base + 2 * k * _CPB
      idesc = [
          pltpu.async_copy(src_hbm.at[pl.ds(crow, _CPB)], src_i[0], sem_i[0]),
          pltpu.async_copy(dst_hbm.at[pl.ds(crow, _CPB)], dst_i[0], sem_i[1]),
          pltpu.async_copy(src_hbm.at[pl.ds(crow + _CPB, _CPB)], src_i[1],
                           sem_i[2]),
          pltpu.async_copy(dst_hbm.at[pl.ds(crow + _CPB, _CPB)], dst_i[1],
                           sem_i[3]),
      ]
      gdesc = [None] * depth
      sdesc = [None] * depth
      for it in range(body_chunks + lead):
        c = it - lead
        if c >= 0:
          b = c % depth
          gdesc[b].wait()
          p, jc = divmod(c, _CPB)
          sdesc[b] = pltpu.async_copy(
              msgs[b], acc_sh.at[dst_i[p].at[jc]], sem_s[b], add=True
          )
        if it < body_chunks:
          p, j = divmod(it, _CPB)
          if j == 0:
            idesc[2 * p].wait()
            idesc[2 * p + 1].wait()
          b2 = it % depth
          if it >= depth:
            sdesc[b2].wait()
          gdesc[b2] = pltpu.async_copy(
              y_hbm.at[src_i[p].at[j]], msgs[b2], sem_g[b2]
          )
      for b in range(depth):
        sdesc[b].wait()
      return carry

    lax.fori_loop(0, blocks_per_tile // 2, body, 0)
    plsc.subcore_barrier()
    pltpu.sync_copy(
        acc_sh.at[pl.ds(rbase, rows)], out_hbm.at[cid, pl.ds(rbase, rows)]
    )

  return agg_kernel


def _tc_xw(X, W1, blk=2000):
  n, d_in = X.shape
  d_hid = W1.shape[1]

  def body(x_ref, w_ref, y_ref):
    y_ref[...] = jnp.dot(
        x_ref[...], w_ref[...], preferred_element_type=jnp.float32)

  return pl.pallas_call(
      body,
      grid=(n // blk,),
      in_specs=[
          pl.BlockSpec((blk, d_in), lambda i: (i, 0)),
          pl.BlockSpec((d_in, d_hid), lambda i: (0, 0)),
      ],
      out_specs=pl.BlockSpec((blk, d_hid), lambda i: (i, 0)),
      out_shape=jax.ShapeDtypeStruct((n, d_hid), jnp.float32),
  )(X, W1)


def _tc_scale(xw, degp, blk=2000):
  n, d_hid = xw.shape

  def body(xw_ref, d_ref, y_ref, dinv_ref):
    deg = d_ref[0][:, 0:1] + d_ref[1][:, 0:1] + 1.0
    dinv = lax.rsqrt(deg)
    y_ref[...] = xw_ref[...] * dinv
    dinv_ref[...] = dinv

  return pl.pallas_call(
      body,
      grid=(n // blk,),
      in_specs=[
          pl.BlockSpec((blk, d_hid), lambda i: (i, 0)),
          pl.BlockSpec((2, blk, _L), lambda i: (0, i, 0)),
      ],
      out_specs=[
          pl.BlockSpec((blk, d_hid), lambda i: (i, 0)),
          pl.BlockSpec((blk, 1), lambda i: (i, 0)),
      ],
      out_shape=[
          jax.ShapeDtypeStruct((n, d_hid), jnp.float32),
          jax.ShapeDtypeStruct((n, 1), jnp.float32),
      ],
  )(xw, degp)


def _tc_mid(aggp, y, dinv, b1, W2, blk=2000):
  n, d_hid = y.shape
  d_out = W2.shape[1]

  def body(p_ref, y_ref, dinv_ref, b1_ref, w2_ref, y2_ref):
    dinv = dinv_ref[...]
    pre = dinv * (p_ref[0] + p_ref[1] + y_ref[...]) + b1_ref[...]
    h = jnp.maximum(pre, 0.0)
    z = jnp.dot(h, w2_ref[...], preferred_element_type=jnp.float32)
    y2_ref[...] = z * dinv

  return pl.pallas_call(
      body,
      grid=(n // blk,),
      in_specs=[
          pl.BlockSpec((2, blk, d_hid), lambda i: (0, i, 0)),
          pl.BlockSpec((blk, d_hid), lambda i: (i, 0)),
          pl.BlockSpec((blk, 1), lambda i: (i, 0)),
          pl.BlockSpec((1, d_hid), lambda i: (0, 0)),
          pl.BlockSpec((d_hid, d_out), lambda i: (0, 0)),
      ],
      out_specs=pl.BlockSpec((blk, d_out), lambda i: (i, 0)),
      out_shape=jax.ShapeDtypeStruct((n, d_out), jnp.float32),
  )(aggp, y, dinv, b1.reshape(1, -1), W2)


def _tc_final(aggp, y2, dinv, b2, blk=2000):
  n, d_out = y2.shape

  def body(q_ref, y2_ref, dinv_ref, b2_ref, out_ref):
    o = dinv_ref[...] * (q_ref[0] + q_ref[1] + y2_ref[...]) + b2_ref[...]
    m = jnp.max(o, axis=1, keepdims=True)
    e = jnp.exp(o - m)
    s = jnp.sum(e, axis=1, keepdims=True)
    out_ref[...] = (o - m) - jnp.log(s)

  return pl.pallas_call(
      body,
      grid=(n // blk,),
      in_specs=[
          pl.BlockSpec((2, blk, d_out), lambda i: (0, i, 0)),
          pl.BlockSpec((blk, d_out), lambda i: (i, 0)),
          pl.BlockSpec((blk, 1), lambda i: (i, 0)),
          pl.BlockSpec((1, d_out), lambda i: (0, 0)),
      ],
      out_specs=pl.BlockSpec((blk, d_out), lambda i: (i, 0)),
      out_shape=jax.ShapeDtypeStruct((n, d_out), jnp.float32),
  )(aggp, y2, dinv, b2.reshape(1, -1))


def kernel(X, edge_index, W1, b1, W2, b2):
  n, _ = X.shape
  e = edge_index.shape[1]

  src = edge_index[0].astype(jnp.int32)
  dst = edge_index[1].astype(jnp.int32)

  blk_edges = _NW * _CHUNK * _CPB
  bpt = 2 * -(-e // (2 * blk_edges))
  e_pad = bpt * blk_edges
  n_pad = (n + _NS) // _NS * _NS
  if e_pad > e:
    pad = e_pad - e
    src = jnp.concatenate([src, jnp.arange(pad, dtype=jnp.int32) % n])
    dst = jnp.concatenate(
        [dst, n + jnp.arange(pad, dtype=jnp.int32) % (n_pad - n)])
  nrows = e_pad // _CHUNK
  src2d = src.reshape(nrows, _CHUNK)
  dst2d = dst.reshape(nrows, _CHUNK)

  d_hid = W1.shape[1]
  d_out = W2.shape[1]
  degp = _make_deg_kernel(n_pad, bpt)(dst2d)
  xw = _tc_xw(X, W1)
  y1, dinv = _tc_scale(xw, degp)
  agg1 = _make_agg_kernel(n_pad, d_hid, bpt, 2, 1)(y1, src2d, dst2d)
  y2 = _tc_mid(agg1, y1, dinv, b1, W2)
  agg2 = _make_agg_kernel(n_pad, d_out, bpt, 4, 2)(y2, src2d, dst2d)
  return _tc_final(agg2, y2, dinv, b2)


# speedup vs baseline: 2.6009x; 1.0106x over previous
import functools

import jax
import jax.numpy as jnp
from jax import lax
from jax.experimental import pallas as pl
from jax.experimental.pallas import tpu as pltpu
from jax.experimental.pallas import tpu_sc as plsc

_NC = 2
_NS = 16
_NW = _NC * _NS
_L = 16
_CHUNK = 128
_CPB = 8


def _fill_vmem(ref, rows, cols, value):
  vec = jnp.full((_L,), value, jnp.float32)

  def body(i, c):
    for j in range(cols // _L):
      ref[i, pl.ds(j * _L, _L)] = vec
    return c

  lax.fori_loop(0, rows, body, 0)


def _zero_shared_rows(src_v, acc_sh, rbase, rows):
  nfull, rem = divmod(rows, _CHUNK)
  for k in range(nfull):
    pltpu.sync_copy(src_v, acc_sh.at[pl.ds(rbase + k * _CHUNK, _CHUNK)])
  if rem:
    pltpu.sync_copy(
        src_v.at[pl.ds(0, rem)],
        acc_sh.at[pl.ds(rbase + nfull * _CHUNK, rem)],
    )


def _make_deg_kernel(n_pad, blocks_per_tile):
  rows = n_pad // _NS

  assert blocks_per_tile % 2 == 0

  @functools.partial(
      pl.kernel,
      out_type=jax.ShapeDtypeStruct((_NC, n_pad, _L), jnp.float32),
      mesh=plsc.VectorSubcoreMesh(core_axis_name="c", subcore_axis_name="s"),
      compiler_params=pltpu.CompilerParams(use_tc_tiling_on_sc=False),
      scratch_types=[
          [pltpu.VMEM((_CPB, _CHUNK), jnp.int32) for _ in range(2)],
          pltpu.VMEM((_CHUNK, _L), jnp.float32),
          pltpu.VMEM_SHARED((n_pad, _L), jnp.float32),
          [pltpu.SemaphoreType.DMA for _ in range(2)],
      ],
  )
  def deg_kernel(dst_hbm, out_hbm, dst_i, val_v, acc_sh, sem_i):
    cid = lax.axis_index("c")
    sid = lax.axis_index("s")
    wid = sid * _NC + cid
    rbase = sid * rows

    _fill_vmem(val_v, _CHUNK, _L, 0.0)
    _zero_shared_rows(val_v, acc_sh, rbase, rows)
    plsc.subcore_barrier()

    _fill_vmem(val_v, _CHUNK, _L, 1.0)
    bbase = wid * blocks_per_tile

    def outer(t, c):
      crow = (bbase + 2 * t) * _CPB
      da = pltpu.async_copy(dst_hbm.at[pl.ds(crow, _CPB)], dst_i[0], sem_i[0])
      db = pltpu.async_copy(
          dst_hbm.at[pl.ds(crow + _CPB, _CPB)], dst_i[1], sem_i[1])
      da.wait()
      for j in range(_CPB):
        pltpu.sync_copy(val_v, acc_sh.at[dst_i[0].at[j]], add=True)
      db.wait()
      for j in range(_CPB):
        pltpu.sync_copy(val_v, acc_sh.at[dst_i[1].at[j]], add=True)
      return c

    lax.fori_loop(0, blocks_per_tile // 2, outer, 0)
    plsc.subcore_barrier()
    pltpu.sync_copy(
        acc_sh.at[pl.ds(rbase, rows)], out_hbm.at[cid, pl.ds(rbase, rows)]
    )

  return deg_kernel


def _make_agg_kernel(n_pad, d, blocks_per_tile, depth, lead):
  rows = n_pad // _NS
  assert blocks_per_tile % 2 == 0
  assert lead % depth != 0 and (depth - lead) % depth != 0
  body_chunks = 2 * _CPB

  @functools.partial(
      pl.kernel,
      out_type=jax.ShapeDtypeStruct((_NC, n_pad, d), jnp.float32),
      mesh=plsc.VectorSubcoreMesh(core_axis_name="c", subcore_axis_name="s"),
      compiler_params=pltpu.CompilerParams(use_tc_tiling_on_sc=False),
      scratch_types=[
          [pltpu.VMEM((_CPB, _CHUNK), jnp.int32) for _ in range(2)],
          [pltpu.VMEM((_CPB, _CHUNK), jnp.int32) for _ in range(2)],
          [pltpu.VMEM((_CHUNK, d), jnp.float32) for _ in range(depth)],
          pltpu.VMEM_SHARED((n_pad, d), jnp.float32),
          [pltpu.SemaphoreType.DMA for _ in range(depth)],
          [pltpu.SemaphoreType.DMA for _ in range(depth)],
          [pltpu.SemaphoreType.DMA for _ in range(4)],
      ],
  )
  def agg_kernel(y_hbm, src_hbm, dst_hbm, out_hbm, src_i, dst_i, msgs,
                 acc_sh, sem_g, sem_s, sem_i):
    cid = lax.axis_index("c")
    sid = lax.axis_index("s")
    wid = sid * _NC + cid
    rbase = sid * rows

    _fill_vmem(msgs[0], _CHUNK, d, 0.0)
    _zero_shared_rows(msgs[0], acc_sh, rbase, rows)
    plsc.subcore_barrier()

    base = wid * blocks_per_tile * _CPB

    def body(k, carry):
      crow = base + 2 * k * _CPB
      idesc = [
          pltpu.async_copy(src_hbm.at[pl.ds(crow, _CPB)], src_i[0], sem_i[0]),
          pltpu.async_copy(dst_hbm.at[pl.ds(crow, _CPB)], dst_i[0], sem_i[1]),
          pltpu.async_copy(src_hbm.at[pl.ds(crow + _CPB, _CPB)], src_i[1],
                           sem_i[2]),
          pltpu.async_copy(dst_hbm.at[pl.ds(crow + _CPB, _CPB)], dst_i[1],
                           sem_i[3]),
      ]
      gdesc = [None] * depth
      sdesc = [None] * depth
      for it in range(body_chunks + lead):
        c = it - lead
        if c >= 0:
          b = c % depth
          gdesc[b].wait()
          p, jc = divmod(c, _CPB)
          sdesc[b] = pltpu.async_copy(
              msgs[b], acc_sh.at[dst_i[p].at[jc]], sem_s[b], add=True
          )
        if it < body_chunks:
          p, j = divmod(it, _CPB)
          if j == 0:
            idesc[2 * p].wait()
            idesc[2 * p + 1].wait()
          b2 = it % depth
          if it >= depth:
            sdesc[b2].wait()
          gdesc[b2] = pltpu.async_copy(
              y_hbm.at[src_i[p].at[j]], msgs[b2], sem_g[b2]
          )
      for b in range(depth):
        sdesc[b].wait()
      return carry

    lax.fori_loop(0, blocks_per_tile // 2, body, 0)
    plsc.subcore_barrier()
    pltpu.sync_copy(
        acc_sh.at[pl.ds(rbase, rows)], out_hbm.at[cid, pl.ds(rbase, rows)]
    )

  return agg_kernel


def _tc_xw(X, W1, blk=2000):
  n, d_in = X.shape
  d_hid = W1.shape[1]

  def body(x_ref, w_ref, y_ref):
    y_ref[...] = jnp.dot(
        x_ref[...], w_ref[...], preferred_element_type=jnp.float32)

  return pl.pallas_call(
      body,
      grid=(n // blk,),
      in_specs=[
          pl.BlockSpec((blk, d_in), lambda i: (i, 0)),
          pl.BlockSpec((d_in, d_hid), lambda i: (0, 0)),
      ],
      out_specs=pl.BlockSpec((blk, d_hid), lambda i: (i, 0)),
      out_shape=jax.ShapeDtypeStruct((n, d_hid), jnp.float32),
  )(X, W1)


def _tc_scale(xw, degp, blk=2000):
  n, d_hid = xw.shape

  def body(xw_ref, d_ref, y_ref, dinv_ref):
    deg = d_ref[0][:, 0:1] + d_ref[1][:, 0:1] + 1.0
    dinv = lax.rsqrt(deg)
    y_ref[...] = xw_ref[...] * dinv
    dinv_ref[...] = dinv

  return pl.pallas_call(
      body,
      grid=(n // blk,),
      in_specs=[
          pl.BlockSpec((blk, d_hid), lambda i: (i, 0)),
          pl.BlockSpec((2, blk, _L), lambda i: (0, i, 0)),
      ],
      out_specs=[
          pl.BlockSpec((blk, d_hid), lambda i: (i, 0)),
          pl.BlockSpec((blk, 1), lambda i: (i, 0)),
      ],
      out_shape=[
          jax.ShapeDtypeStruct((n, d_hid), jnp.float32),
          jax.ShapeDtypeStruct((n, 1), jnp.float32),
      ],
  )(xw, degp)


def _tc_mid(aggp, y, dinv, b1, W2, blk=2000):
  n, d_hid = y.shape
  d_out = W2.shape[1]

  def body(p_ref, y_ref, dinv_ref, b1_ref, w2_ref, y2_ref):
    dinv = dinv_ref[...]
    pre = dinv * (p_ref[0] + p_ref[1] + y_ref[...]) + b1_ref[...]
    h = jnp.maximum(pre, 0.0)
    z = jnp.dot(h, w2_ref[...], preferred_element_type=jnp.float32)
    y2_ref[...] = z * dinv

  return pl.pallas_call(
      body,
      grid=(n // blk,),
      in_specs=[
          pl.BlockSpec((2, blk, d_hid), lambda i: (0, i, 0)),
          pl.BlockSpec((blk, d_hid), lambda i: (i, 0)),
          pl.BlockSpec((blk, 1), lambda i: (i, 0)),
          pl.BlockSpec((1, d_hid), lambda i: (0, 0)),
          pl.BlockSpec((d_hid, d_out), lambda i: (0, 0)),
      ],
      out_specs=pl.BlockSpec((blk, d_out), lambda i: (i, 0)),
      out_shape=jax.ShapeDtypeStruct((n, d_out), jnp.float32),
  )(aggp, y, dinv, b1.reshape(1, -1), W2)


def _tc_final(aggp, y2, dinv, b2, blk=2000):
  n, d_out = y2.shape

  def body(q_ref, y2_ref, dinv_ref, b2_ref, out_ref):
    o = dinv_ref[...] * (q_ref[0] + q_ref[1] + y2_ref[...]) + b2_ref[...]
    m = jnp.max(o, axis=1, keepdims=True)
    e = jnp.exp(o - m)
    s = jnp.sum(e, axis=1, keepdims=True)
    out_ref[...] = (o - m) - jnp.log(s)

  return pl.pallas_call(
      body,
      grid=(n // blk,),
      in_specs=[
          pl.BlockSpec((2, blk, d_out), lambda i: (0, i, 0)),
          pl.BlockSpec((blk, d_out), lambda i: (i, 0)),
          pl.BlockSpec((blk, 1), lambda i: (i, 0)),
          pl.BlockSpec((1, d_out), lambda i: (0, 0)),
      ],
      out_specs=pl.BlockSpec((blk, d_out), lambda i: (i, 0)),
      out_shape=jax.ShapeDtypeStruct((n, d_out), jnp.float32),
  )(aggp, y2, dinv, b2.reshape(1, -1))


def kernel(X, edge_index, W1, b1, W2, b2):
  n, _ = X.shape
  e = edge_index.shape[1]

  src = edge_index[0].astype(jnp.int32)
  dst = edge_index[1].astype(jnp.int32)

  blk_edges = _NW * _CHUNK * _CPB
  bpt = 2 * -(-e // (2 * blk_edges))
  e_pad = bpt * blk_edges
  n_pad = (n + _NS) // _NS * _NS
  if e_pad > e:
    pad = e_pad - e
    src = jnp.concatenate([src, jnp.arange(pad, dtype=jnp.int32) % n])
    dst = jnp.concatenate(
        [dst, n + jnp.arange(pad, dtype=jnp.int32) % (n_pad - n)])
  nrows = e_pad // _CHUNK
  src2d = src.reshape(nrows, _CHUNK)
  dst2d = dst.reshape(nrows, _CHUNK)

  d_hid = W1.shape[1]
  d_out = W2.shape[1]
  degp = _make_deg_kernel(n_pad, bpt)(dst2d)
  xw = _tc_xw(X, W1)
  y1, dinv = _tc_scale(xw, degp)
  agg1 = _make_agg_kernel(n_pad, d_hid, bpt, 2, 1)(y1, src2d, dst2d)
  y2 = _tc_mid(agg1, y1, dinv, b1, W2)
  agg2 = _make_agg_kernel(n_pad, d_out, bpt, 4, 2)(y2, src2d, dst2d)
  return _tc_final(agg2, y2, dinv, b2)
